# Initial kernel scaffold; baseline (speedup 1.0000x reference)
import jax
import jax.numpy as jnp
from jax.experimental import pallas as pl


def kernel(src, mode, time, weekday, emb_loc, emb_mode, minute_embed, hour_embed, weekday_embed):
    raise NotImplementedError("write your pallas kernel here")


# speedup vs baseline: 4.8447x; 4.8447x over previous
import functools
import math

import jax
import jax.numpy as jnp
import numpy as np
from jax import lax
from jax.experimental import pallas as pl
from jax.experimental.pallas import tpu as pltpu
from jax.experimental.pallas import tpu_sc as plsc

_MINUTE_SIZE = 4
_HOUR_SIZE = 24

_NC = 2
_NS = 16
_NW = _NC * _NS


def _pos_table(emb_size, maxlen):
    den = np.exp(-np.arange(0, emb_size, 2, dtype=np.float64) * math.log(10000.0) / emb_size)
    pos = np.arange(0, maxlen, dtype=np.float64).reshape(maxlen, 1)
    pe = np.zeros((maxlen, emb_size), dtype=np.float32)
    pe[:, 0::2] = np.sin(pos * den).astype(np.float32)
    pe[:, 1::2] = np.cos(pos * den).astype(np.float32)
    return jnp.asarray(pe)


def kernel(src, mode, time, weekday, emb_loc, emb_mode, minute_embed, hour_embed, weekday_embed):
    S, B = src.shape
    V, D = emb_loc.shape
    scale = math.sqrt(D)
    n_time = _MINUTE_SIZE * _HOUR_SIZE
    n_wk = weekday_embed.shape[0]

    t_time = (jnp.repeat(hour_embed, _MINUTE_SIZE, axis=0)
              + jnp.tile(minute_embed, (_HOUR_SIZE, 1)))
    t2 = (emb_mode[:, None, None, :] + t_time[None, :, None, :]
          + weekday_embed[None, None, :, :]) * scale
    t2 = t2.reshape(-1, D)

    pe = _pos_table(D, S)

    CB = B // _NW
    NG = CB // 16

    mesh = plsc.VectorSubcoreMesh(core_axis_name="c", subcore_axis_name="s",
                                  num_cores=_NC, num_subcores=_NS)

    @functools.partial(
        pl.kernel,
        out_type=jax.ShapeDtypeStruct((S, B, D), jnp.float32),
        mesh=mesh,
        compiler_params=pltpu.CompilerParams(use_tc_tiling_on_sc=False),
        scratch_types=[
            pltpu.VMEM((CB,), jnp.int32),
            pltpu.VMEM((CB,), jnp.int32),
            pltpu.VMEM((CB,), jnp.int32),
            pltpu.VMEM((CB,), jnp.int32),
            pltpu.VMEM((CB,), jnp.int32),
            pltpu.VMEM((CB, D), jnp.float32),
            pltpu.VMEM((CB, D), jnp.float32),
            pltpu.VMEM((D,), jnp.float32),
            pltpu.SemaphoreType.DMA,
            pltpu.SemaphoreType.DMA,
        ],
    )
    def _sc_kernel(src_h, mode_h, time_h, wk_h, loc_h, t2_h, pe_h, out_h,
                   src_v, mode_v, time_v, wk_v, ci_v, rows_v, small_v, pe_v,
                   sem0, sem1):
        wid = lax.axis_index("s") * _NC + lax.axis_index("c")
        base = wid * CB

        def row_body(s, carry):
            pltpu.sync_copy(src_h.at[s, pl.ds(base, CB)], src_v)
            pltpu.sync_copy(mode_h.at[s, pl.ds(base, CB)], mode_v)
            pltpu.sync_copy(time_h.at[s, pl.ds(base, CB)], time_v)
            pltpu.sync_copy(wk_h.at[s, pl.ds(base, CB)], wk_v)
            pltpu.sync_copy(pe_h.at[s], pe_v)
            for g in range(NG):
                sl = pl.ds(g * 16, 16)
                ci_v[sl] = (mode_v[sl] * n_time + time_v[sl]) * n_wk + wk_v[sl]
            cp0 = pltpu.async_copy(loc_h.at[src_v], rows_v, sem0)
            cp1 = pltpu.async_copy(t2_h.at[ci_v], small_v, sem1)
            cp0.wait()
            cp1.wait()
            pe_regs = [pe_v[pl.ds(j * 16, 16)] for j in range(D // 16)]

            def tok_body(i, c):
                for j in range(D // 16):
                    sl = pl.ds(j * 16, 16)
                    rows_v[i, sl] = rows_v[i, sl] * scale + small_v[i, sl] + pe_regs[j]
                return c

            lax.fori_loop(0, CB, tok_body, 0, unroll=2)
            pltpu.sync_copy(rows_v, out_h.at[s, pl.ds(base, CB)])
            return carry

        lax.fori_loop(0, S, row_body, 0)

    return _sc_kernel(src, mode, time, weekday, emb_loc, t2, pe)


# speedup vs baseline: 7.4431x; 1.5364x over previous
import functools
import math

import jax
import jax.numpy as jnp
import numpy as np
from jax import lax
from jax.experimental import pallas as pl
from jax.experimental.pallas import tpu as pltpu
from jax.experimental.pallas import tpu_sc as plsc

_MINUTE_SIZE = 4
_HOUR_SIZE = 24

_NC = 2
_NS = 16
_NW = _NC * _NS
_NB = 4


def _pos_table(emb_size, maxlen):
    den = np.exp(-np.arange(0, emb_size, 2, dtype=np.float64) * math.log(10000.0) / emb_size)
    pos = np.arange(0, maxlen, dtype=np.float64).reshape(maxlen, 1)
    pe = np.zeros((maxlen, emb_size), dtype=np.float32)
    pe[:, 0::2] = np.sin(pos * den).astype(np.float32)
    pe[:, 1::2] = np.cos(pos * den).astype(np.float32)
    return jnp.asarray(pe)


def kernel(src, mode, time, weekday, emb_loc, emb_mode, minute_embed, hour_embed, weekday_embed):
    S, B = src.shape
    V, D = emb_loc.shape
    scale = math.sqrt(D)
    n_time = _MINUTE_SIZE * _HOUR_SIZE
    n_wk = weekday_embed.shape[0]

    t_time = (jnp.repeat(hour_embed, _MINUTE_SIZE, axis=0)
              + jnp.tile(minute_embed, (_HOUR_SIZE, 1)))
    t2 = (emb_mode[:, None, None, :] + t_time[None, :, None, :]
          + weekday_embed[None, None, :, :]) * scale
    t2 = t2.reshape(-1, D)

    pe = _pos_table(D, S)

    CB = B // _NW
    NG = CB // 16
    NJ = D // 16

    mesh = plsc.VectorSubcoreMesh(core_axis_name="c", subcore_axis_name="s",
                                  num_cores=_NC, num_subcores=_NS)

    @functools.partial(
        pl.kernel,
        out_type=jax.ShapeDtypeStruct((S, B, D), jnp.float32),
        mesh=mesh,
        compiler_params=pltpu.CompilerParams(use_tc_tiling_on_sc=False),
        scratch_types=[
            pltpu.VMEM((_NB, CB), jnp.int32),
            pltpu.VMEM((_NB, CB), jnp.int32),
            pltpu.VMEM((_NB, CB), jnp.int32),
            pltpu.VMEM((_NB, CB), jnp.int32),
            pltpu.VMEM((_NB, CB), jnp.int32),
            pltpu.VMEM((_NB, CB, D), jnp.float32),
            pltpu.VMEM((_NB, CB, D), jnp.float32),
            pltpu.VMEM((S, D), jnp.float32),
            pltpu.SemaphoreType.DMA((_NB,)),
            pltpu.SemaphoreType.DMA((_NB,)),
            pltpu.SemaphoreType.DMA((_NB,)),
        ],
    )
    def _sc_kernel(src_h, mode_h, time_h, wk_h, loc_h, t2_h, pe_h, out_h,
                   src_v, mode_v, time_v, wk_v, ci_v, rows_v, small_v, pe_all,
                   sem_idx, sem_g, sem_out):
        wid = lax.axis_index("s") * _NC + lax.axis_index("c")
        base = wid * CB
        arrs = [(src_h, src_v), (mode_h, mode_v), (time_h, time_v), (wk_h, wk_v)]

        def issue_idx(g, b):
            for h, v in arrs:
                pltpu.async_copy(h.at[g, pl.ds(base, CB)], v.at[b], sem_idx.at[b])

        def wait_idx(g, b):
            for h, v in arrs:
                pltpu.make_async_copy(h.at[g, pl.ds(base, CB)], v.at[b], sem_idx.at[b]).wait()

        def compute_ci(b):
            for gg in range(NG):
                sl = pl.ds(gg * 16, 16)
                ci_v[b, sl] = (mode_v[b, sl] * n_time + time_v[b, sl]) * n_wk + wk_v[b, sl]

        def issue_gather(b):
            pltpu.async_copy(loc_h.at[src_v.at[b]], rows_v.at[b], sem_g.at[b])
            pltpu.async_copy(t2_h.at[ci_v.at[b]], small_v.at[b], sem_g.at[b])

        def wait_gather(b):
            pltpu.make_async_copy(loc_h.at[src_v.at[b]], rows_v.at[b], sem_g.at[b]).wait()
            pltpu.make_async_copy(t2_h.at[ci_v.at[b]], small_v.at[b], sem_g.at[b]).wait()

        def issue_out(g, b):
            pltpu.async_copy(rows_v.at[b], out_h.at[g, pl.ds(base, CB)], sem_out.at[b])

        def wait_out(g, b):
            pltpu.make_async_copy(rows_v.at[b], out_h.at[g, pl.ds(base, CB)], sem_out.at[b]).wait()

        def compute(g, b):
            pe_regs = [pe_all[g, pl.ds(j * 16, 16)] for j in range(NJ)]

            def tok_body(i, c):
                for j in range(NJ):
                    sl = pl.ds(j * 16, 16)
                    rows_v[b, i, sl] = (rows_v[b, i, sl] * scale
                                        + small_v[b, i, sl] + pe_regs[j])
                return c

            lax.fori_loop(0, CB, tok_body, 0, unroll=2)

        pltpu.sync_copy(pe_h, pe_all)
        for b in range(_NB):
            issue_idx(b, b)
        for b in range(_NB):
            wait_idx(b, b)
            compute_ci(b)
            issue_gather(b)

        def body(G, c):
            for b in range(_NB):
                g = G * _NB + b
                wait_gather(b)

                @pl.when(g + _NB < S)
                def _():
                    issue_idx(g + _NB, b)

                compute(g, b)
                issue_out(g, b)
            for b in range(_NB):
                g = G * _NB + b

                @pl.when(g + _NB < S)
                def _():
                    wait_idx(g + _NB, b)
                    compute_ci(b)
                    wait_out(g, b)
                    issue_gather(b)

            return c

        lax.fori_loop(0, S // _NB, body, 0)
        for b in range(_NB):
            wait_out(S - _NB + b, b)

    return _sc_kernel(src, mode, time, weekday, emb_loc, t2, pe)


# speedup vs baseline: 8.7183x; 1.1713x over previous
import functools
import math

import jax
import jax.numpy as jnp
import numpy as np
from jax import lax
from jax.experimental import pallas as pl
from jax.experimental.pallas import tpu as pltpu
from jax.experimental.pallas import tpu_sc as plsc

_MINUTE_SIZE = 4
_HOUR_SIZE = 24

_NC = 2
_NS = 16
_NW = _NC * _NS
_NB = 4


def _pos_table(emb_size, maxlen):
    den = np.exp(-np.arange(0, emb_size, 2, dtype=np.float64) * math.log(10000.0) / emb_size)
    pos = np.arange(0, maxlen, dtype=np.float64).reshape(maxlen, 1)
    pe = np.zeros((maxlen, emb_size), dtype=np.float32)
    pe[:, 0::2] = np.sin(pos * den).astype(np.float32)
    pe[:, 1::2] = np.cos(pos * den).astype(np.float32)
    return jnp.asarray(pe)


def kernel(src, mode, time, weekday, emb_loc, emb_mode, minute_embed, hour_embed, weekday_embed):
    S, B = src.shape
    V, D = emb_loc.shape
    scale = math.sqrt(D)
    n_time = _MINUTE_SIZE * _HOUR_SIZE
    n_wk = weekday_embed.shape[0]

    t_time = (jnp.repeat(hour_embed, _MINUTE_SIZE, axis=0)
              + jnp.tile(minute_embed, (_HOUR_SIZE, 1)))
    t2 = (emb_mode[:, None, None, :] + t_time[None, :, None, :]
          + weekday_embed[None, None, :, :]) * scale
    t2 = t2.reshape(-1, D)

    pe = _pos_table(D, S)

    CB = B // _NW
    NG = CB // 16
    NJ = D // 16

    mesh = plsc.VectorSubcoreMesh(core_axis_name="c", subcore_axis_name="s",
                                  num_cores=_NC, num_subcores=_NS)

    @functools.partial(
        pl.kernel,
        out_type=jax.ShapeDtypeStruct((S, B, D), jnp.float32),
        mesh=mesh,
        compiler_params=pltpu.CompilerParams(use_tc_tiling_on_sc=False),
        scratch_types=[
            pltpu.VMEM((_NB, CB), jnp.int32),
            pltpu.VMEM((_NB, CB), jnp.int32),
            pltpu.VMEM((_NB, CB), jnp.int32),
            pltpu.VMEM((_NB, CB), jnp.int32),
            pltpu.VMEM((_NB, CB), jnp.int32),
            pltpu.VMEM((_NB, CB, D), jnp.float32),
            pltpu.VMEM((_NB, CB, D), jnp.float32),
            pltpu.VMEM((S, D), jnp.float32),
            pltpu.SemaphoreType.DMA((_NB,)),
            pltpu.SemaphoreType.DMA((_NB,)),
            pltpu.SemaphoreType.DMA((_NB,)),
        ],
    )
    def _sc_kernel(src_h, mode_h, time_h, wk_h, loc_h, t2_h, pe_h, out_h,
                   src_v, mode_v, time_v, wk_v, ci_v, rows_v, small_v, pe_all,
                   sem_idx, sem_g, sem_out):
        wid = lax.axis_index("s") * _NC + lax.axis_index("c")
        base = wid * CB
        arrs = [(src_h, src_v), (mode_h, mode_v), (time_h, time_v), (wk_h, wk_v)]

        def issue_idx(g, b):
            for h, v in arrs:
                pltpu.async_copy(h.at[g, pl.ds(base, CB)], v.at[b], sem_idx.at[b])

        def wait_idx(g, b):
            for h, v in arrs:
                pltpu.make_async_copy(h.at[g, pl.ds(base, CB)], v.at[b], sem_idx.at[b]).wait()

        def compute_ci(b):
            for gg in range(NG):
                sl = pl.ds(gg * 16, 16)
                ci_v[b, sl] = (mode_v[b, sl] * n_time + time_v[b, sl]) * n_wk + wk_v[b, sl]

        def issue_gather(b):
            pltpu.async_copy(loc_h.at[src_v.at[b]], rows_v.at[b], sem_g.at[b])
            pltpu.async_copy(t2_h.at[ci_v.at[b]], small_v.at[b], sem_g.at[b])

        def wait_gather(b):
            pltpu.make_async_copy(loc_h.at[src_v.at[b]], rows_v.at[b], sem_g.at[b]).wait()
            pltpu.make_async_copy(t2_h.at[ci_v.at[b]], small_v.at[b], sem_g.at[b]).wait()

        def issue_out(g, b):
            pltpu.async_copy(rows_v.at[b], out_h.at[g, pl.ds(base, CB)], sem_out.at[b])

        def wait_out(g, b):
            pltpu.make_async_copy(rows_v.at[b], out_h.at[g, pl.ds(base, CB)], sem_out.at[b]).wait()

        def compute(g, b):
            pe_regs = [pe_all[g, pl.ds(j * 16, 16)] for j in range(NJ)]

            @plsc.parallel_loop(0, CB, step=1, unroll=4)
            def tok_body(i):
                sls = [pl.ds(j * 16, 16) for j in range(NJ)]
                r = [rows_v[b, i, sl] for sl in sls]
                sm = [small_v[b, i, sl] for sl in sls]
                for j, sl in enumerate(sls):
                    rows_v[b, i, sl] = r[j] * scale + (sm[j] + pe_regs[j])

        pltpu.sync_copy(pe_h, pe_all)
        for b in range(_NB):
            issue_idx(b, b)
        for b in range(_NB):
            wait_idx(b, b)
            compute_ci(b)
            issue_gather(b)

        def body(G, c):
            for b in range(_NB):
                g = G * _NB + b
                wait_gather(b)

                @pl.when(g + _NB < S)
                def _():
                    issue_idx(g + _NB, b)

                compute(g, b)
                issue_out(g, b)
            for b in range(_NB):
                g = G * _NB + b

                @pl.when(g + _NB < S)
                def _():
                    wait_idx(g + _NB, b)
                    compute_ci(b)
                    wait_out(g, b)
                    issue_gather(b)

            return c

        lax.fori_loop(0, S // _NB, body, 0)
        for b in range(_NB):
            wait_out(S - _NB + b, b)

    return _sc_kernel(src, mode, time, weekday, emb_loc, t2, pe)
